# baseline (speedup 1.0000x reference)
import jax
import jax.numpy as jnp
from jax.experimental import pallas as pl
from jax.experimental.pallas import tpu as pltpu

_B = 64


def _fused_body(e_ref, seg_ref, wc_ref, bc_ref, wa_ref, out_ref,
                acc_ref, d_ref):
    i = pl.program_id(0)
    nsteps = pl.num_programs(0)

    @pl.when(i == 0)
    def _init():
        acc_ref[...] = jnp.zeros_like(acc_ref)
        d_ref[...] = jnp.zeros_like(d_ref)

    e = e_ref[...]
    e16 = e.astype(jnp.bfloat16)
    t = jnp.tanh(jnp.dot(e16, wc_ref[...].astype(jnp.bfloat16),
                         preferred_element_type=jnp.float32) + bc_ref[...])
    s = jnp.dot(t.astype(jnp.bfloat16), wa_ref[...].astype(jnp.bfloat16),
                preferred_element_type=jnp.float32)
    ex = jnp.exp(s)

    seg = seg_ref[0].astype(jnp.float32)
    blk = seg.shape[0]
    bids = jax.lax.broadcasted_iota(jnp.int32, (blk, _B), 1).astype(jnp.float32)
    x16 = jnp.where(bids == seg, ex, 0.0).astype(jnp.bfloat16)

    ones16 = jnp.ones((blk, 1), dtype=jnp.bfloat16)
    d_ref[...] += jax.lax.dot_general(
        x16, ones16, (((0,), (0,)), ((), ())),
        preferred_element_type=jnp.float32)
    acc_ref[...] += jax.lax.dot_general(
        x16, e16, (((0,), (0,)), ((), ())),
        preferred_element_type=jnp.float32)

    @pl.when(i == nsteps - 1)
    def _finish():
        d = d_ref[...]
        out_ref[...] = jnp.where(d > 0, acc_ref[...] / d, 0.0)


def kernel(embeddings, batch, W_c, b_c, w_a, b_a):
    n, d = embeddings.shape
    blk = 5000
    nblk = n // blk
    assert nblk * blk == n
    seg3 = batch.astype(jnp.int32).reshape(nblk, blk, 1)
    bc2 = b_c.reshape(1, d)

    out = pl.pallas_call(
        _fused_body,
        grid=(nblk,),
        in_specs=[
            pl.BlockSpec((blk, d), lambda i: (i, 0)),
            pl.BlockSpec((1, blk, 1), lambda i: (i, 0, 0)),
            pl.BlockSpec((d, d), lambda i: (0, 0)),
            pl.BlockSpec((1, d), lambda i: (0, 0)),
            pl.BlockSpec((d, 1), lambda i: (0, 0)),
        ],
        out_specs=pl.BlockSpec((_B, d), lambda i: (0, 0)),
        out_shape=jax.ShapeDtypeStruct((_B, d), jnp.float32),
        scratch_shapes=[
            pltpu.VMEM((_B, d), jnp.float32),
            pltpu.VMEM((_B, 1), jnp.float32),
        ],
    )(embeddings, seg3, W_c, bc2, w_a)
    return out


# speedup vs baseline: 1.6290x; 1.6290x over previous
import jax
import jax.numpy as jnp
from jax.experimental import pallas as pl
from jax.experimental.pallas import tpu as pltpu

_B = 64


def _fused_body(e_ref, seg_ref, wc_ref, bc_ref, wa_ref, out_ref,
                acc_ref, d_ref, e16_scr, s_scr):
    i = pl.program_id(0)
    nsteps = pl.num_programs(0)
    p = jax.lax.rem(i, 2)
    q = jax.lax.rem(i + 1, 2)

    @pl.when(i == 0)
    def _init():
        acc_ref[...] = jnp.zeros_like(acc_ref)
        d_ref[...] = jnp.zeros_like(d_ref)
        s_scr[pl.ds(1, 1)] = jnp.full(s_scr.shape[1:], -jnp.inf,
                                      dtype=jnp.float32)[None]
        e16_scr[pl.ds(1, 1)] = jnp.zeros(e16_scr.shape[1:],
                                         dtype=jnp.bfloat16)[None]

    e16 = e_ref[...].astype(jnp.bfloat16)
    t = jnp.tanh(jnp.dot(e16, wc_ref[...].astype(jnp.bfloat16),
                         preferred_element_type=jnp.float32) + bc_ref[...])
    sT = jax.lax.dot_general(wa_ref[...].astype(jnp.bfloat16),
                             t.astype(jnp.bfloat16), (((1,), (1,)), ((), ())),
                             preferred_element_type=jnp.float32)
    e16_scr[pl.ds(p, 1)] = e16[None]
    s_scr[pl.ds(p, 1)] = sT[None]

    s_prev = s_scr[pl.ds(q, 1)][0]
    ex = jnp.exp(s_prev)
    seg = seg_ref[0].astype(jnp.float32)
    blk = seg.shape[1]
    bids = jax.lax.broadcasted_iota(jnp.int32, (_B, blk), 0).astype(jnp.float32)
    x16 = jnp.where(bids == seg, ex, 0.0).astype(jnp.bfloat16)
    e16p = e16_scr[pl.ds(q, 1)][0]

    ones16 = jnp.ones((blk, 1), dtype=jnp.bfloat16)
    d_ref[...] += jax.lax.dot_general(
        x16, ones16, (((1,), (0,)), ((), ())),
        preferred_element_type=jnp.float32)
    acc_ref[...] += jax.lax.dot_general(
        x16, e16p, (((1,), (0,)), ((), ())),
        preferred_element_type=jnp.float32)

    @pl.when(i == nsteps - 1)
    def _finish():
        d = d_ref[...]
        out_ref[...] = jnp.where(d > 0, acc_ref[...] / d, 0.0)


def kernel(embeddings, batch, W_c, b_c, w_a, b_a):
    n, d = embeddings.shape
    blk = 5000
    nblk = n // blk
    assert nblk * blk == n
    seg3 = batch.astype(jnp.int32).reshape(nblk, 1, blk)
    bc2 = b_c.reshape(1, d)
    wa2 = w_a.reshape(1, d)

    out = pl.pallas_call(
        _fused_body,
        grid=(nblk + 1,),
        in_specs=[
            pl.BlockSpec((blk, d), lambda i: (jnp.minimum(i, nblk - 1), 0)),
            pl.BlockSpec((1, 1, blk), lambda i: (jnp.maximum(i - 1, 0), 0, 0)),
            pl.BlockSpec((d, d), lambda i: (0, 0)),
            pl.BlockSpec((1, d), lambda i: (0, 0)),
            pl.BlockSpec((1, d), lambda i: (0, 0)),
        ],
        out_specs=pl.BlockSpec((_B, d), lambda i: (0, 0)),
        out_shape=jax.ShapeDtypeStruct((_B, d), jnp.float32),
        scratch_shapes=[
            pltpu.VMEM((_B, d), jnp.float32),
            pltpu.VMEM((_B, 1), jnp.float32),
            pltpu.VMEM((2, blk, d), jnp.bfloat16),
            pltpu.VMEM((2, 1, blk), jnp.float32),
        ],
    )(embeddings, seg3, W_c, bc2, wa2)
    return out


# speedup vs baseline: 1.6392x; 1.0063x over previous
import jax
import jax.numpy as jnp
from jax.experimental import pallas as pl
from jax.experimental.pallas import tpu as pltpu

_B = 64
_CHUNKS = 2


def _fused_body(e_ref, seg_ref, wc_ref, bc_ref, wa_ref, out_ref,
                acc_ref, d_ref):
    i = pl.program_id(0)
    nsteps = pl.num_programs(0)

    @pl.when(i == 0)
    def _init():
        acc_ref[...] = jnp.zeros_like(acc_ref)
        d_ref[...] = jnp.zeros_like(d_ref)

    wc16 = wc_ref[...].astype(jnp.bfloat16)
    wa16 = wa_ref[...].astype(jnp.bfloat16)
    bc = bc_ref[...]
    blk = e_ref.shape[0]
    sub = blk // _CHUNKS
    bids = jax.lax.broadcasted_iota(jnp.int32, (_B, sub), 0).astype(jnp.float32)
    ones16 = jnp.ones((sub, 1), dtype=jnp.bfloat16)

    for c in range(_CHUNKS):
        e16 = e_ref[pl.ds(c * sub, sub), :].astype(jnp.bfloat16)
        t = jnp.tanh(jnp.dot(e16, wc16,
                             preferred_element_type=jnp.float32) + bc)
        sT = jax.lax.dot_general(wa16, t.astype(jnp.bfloat16),
                                 (((1,), (1,)), ((), ())),
                                 preferred_element_type=jnp.float32)
        ex = jnp.exp(sT)
        seg = seg_ref[0, :, pl.ds(c * sub, sub)].astype(jnp.float32)
        x16 = jnp.where(bids == seg, ex, 0.0).astype(jnp.bfloat16)
        d_ref[...] += jax.lax.dot_general(
            x16, ones16, (((1,), (0,)), ((), ())),
            preferred_element_type=jnp.float32)
        acc_ref[...] += jax.lax.dot_general(
            x16, e16, (((1,), (0,)), ((), ())),
            preferred_element_type=jnp.float32)

    @pl.when(i == nsteps - 1)
    def _finish():
        d = d_ref[...]
        out_ref[...] = jnp.where(d > 0, acc_ref[...] / d, 0.0)


def kernel(embeddings, batch, W_c, b_c, w_a, b_a):
    n, d = embeddings.shape
    blk = 5000
    nblk = n // blk
    assert nblk * blk == n
    seg3 = batch.astype(jnp.int32).reshape(nblk, 1, blk)
    bc2 = b_c.reshape(1, d)
    wa2 = w_a.reshape(1, d)

    out = pl.pallas_call(
        _fused_body,
        grid=(nblk,),
        in_specs=[
            pl.BlockSpec((blk, d), lambda i: (i, 0)),
            pl.BlockSpec((1, 1, blk), lambda i: (i, 0, 0)),
            pl.BlockSpec((d, d), lambda i: (0, 0)),
            pl.BlockSpec((1, d), lambda i: (0, 0)),
            pl.BlockSpec((1, d), lambda i: (0, 0)),
        ],
        out_specs=pl.BlockSpec((_B, d), lambda i: (0, 0)),
        out_shape=jax.ShapeDtypeStruct((_B, d), jnp.float32),
        scratch_shapes=[
            pltpu.VMEM((_B, d), jnp.float32),
            pltpu.VMEM((_B, 1), jnp.float32),
        ],
    )(embeddings, seg3, W_c, bc2, wa2)
    return out


# speedup vs baseline: 1.7751x; 1.0829x over previous
import jax
import jax.numpy as jnp
from jax.experimental import pallas as pl
from jax.experimental.pallas import tpu as pltpu

_B = 64


def _fused_body(e_ref, seg_ref, wc_ref, bc_ref, wa_ref, out_ref,
                acc_ref, d_ref):
    i = pl.program_id(0)
    nsteps = pl.num_programs(0)

    @pl.when(i == 0)
    def _init():
        acc_ref[...] = jnp.zeros_like(acc_ref)
        d_ref[...] = jnp.zeros_like(d_ref)

    e = e_ref[...]
    e16 = e.astype(jnp.bfloat16)
    t = jnp.tanh(jnp.dot(e16, wc_ref[...].astype(jnp.bfloat16),
                         preferred_element_type=jnp.float32) + bc_ref[...])
    sT = jax.lax.dot_general(wa_ref[...].astype(jnp.bfloat16),
                             t.astype(jnp.bfloat16), (((1,), (1,)), ((), ())),
                             preferred_element_type=jnp.float32)
    ex = jnp.exp(sT)

    seg = seg_ref[0].astype(jnp.float32)
    blk = seg.shape[1]
    bids = jax.lax.broadcasted_iota(jnp.int32, (_B, blk), 0).astype(jnp.float32)
    x16 = jnp.where(bids == seg, ex, 0.0).astype(jnp.bfloat16)

    ones16 = jnp.ones((blk, 1), dtype=jnp.bfloat16)
    d_ref[...] += jax.lax.dot_general(
        x16, ones16, (((1,), (0,)), ((), ())),
        preferred_element_type=jnp.float32)
    acc_ref[...] += jax.lax.dot_general(
        x16, e16, (((1,), (0,)), ((), ())),
        preferred_element_type=jnp.float32)

    @pl.when(i == nsteps - 1)
    def _finish():
        d = d_ref[...]
        out_ref[...] = jnp.where(d > 0, acc_ref[...] / d, 0.0)


def kernel(embeddings, batch, W_c, b_c, w_a, b_a):
    n, d = embeddings.shape
    blk = 5000
    nblk = n // blk
    assert nblk * blk == n
    seg3 = batch.astype(jnp.int32).reshape(nblk, 1, blk)
    bc2 = b_c.reshape(1, d)
    wa2 = w_a.reshape(1, d)

    out = pl.pallas_call(
        _fused_body,
        grid=(nblk,),
        in_specs=[
            pl.BlockSpec((blk, d), lambda i: (i, 0)),
            pl.BlockSpec((1, 1, blk), lambda i: (i, 0, 0)),
            pl.BlockSpec((d, d), lambda i: (0, 0)),
            pl.BlockSpec((1, d), lambda i: (0, 0)),
            pl.BlockSpec((1, d), lambda i: (0, 0)),
        ],
        out_specs=pl.BlockSpec((_B, d), lambda i: (0, 0)),
        out_shape=jax.ShapeDtypeStruct((_B, d), jnp.float32),
        scratch_shapes=[
            pltpu.VMEM((_B, d), jnp.float32),
            pltpu.VMEM((_B, 1), jnp.float32),
        ],
    )(embeddings, seg3, W_c, bc2, wa2)
    return out
